# Initial kernel scaffold; baseline (speedup 1.0000x reference)
import jax
import jax.numpy as jnp
from jax.experimental import pallas as pl


def kernel(node_features, edge_index, edge_features, edge_vectors, t, pos_emb, params):
    raise NotImplementedError("write your pallas kernel here")


# speedup vs baseline: 48.2961x; 48.2961x over previous
import functools

import numpy as np
import jax
import jax.numpy as jnp
from jax import lax
from jax.experimental import pallas as pl
from jax.experimental.pallas import tpu as pltpu
from jax.experimental.pallas import tpu_sc as plsc

F32 = jnp.float32
I32 = jnp.int32

_NC = 2
_NS = 16
_NW = _NC * _NS
_LANES = 16

_CH = 100

_DIMS = [32, 44, 56, 88]
_TPD = [16, 28, 40, 56]
_KEYD = [8, 14, 20, 28]
_KP = [16, 16, 32, 32]
_VP = [32, 48, 64, 64]
_VW = 64

_NEG = -3.4e38
_SC_PARAMS = pltpu.CompilerParams(use_tc_tiling_on_sc=False,
                                  needs_layout_passes=False)


def _dot(a, b):
  def one(lo, hi):
    return lax.dot_general(
        a[:, lo:hi].astype(jnp.bfloat16), b[lo:hi].astype(jnp.bfloat16),
        (((1,), (0,)), ((), ())), preferred_element_type=F32)
  k = a.shape[1]
  out = one(0, min(k, 256))
  for lo in range(256, k, 256):
    out = out + one(lo, min(k, lo + 256))
  return out


def _layer_sizes(i):
  a = min(i, 3)
  b = min(i + 1, 3)
  return dict(din=_DIMS[a], dout=_DIMS[b], tp=_TPD[a], K=_KEYD[a],
              T=_TPD[b], Kp=_KP[i], Vp=_VP[i],
              Ws=32 + _KP[i] + _VP[i], Wd=32 + _KP[i])


def _table_rows(xn, lp, sizes):
  s = sizes
  xtp = _dot(xn, lp["input_linear"]["w"]) + lp["input_linear"]["b"]
  kn = _dot(xtp, lp["lin_k"]["w"]) + lp["lin_k"]["b"]
  vn = _dot(xtp, lp["lin_v"]["w"]) + lp["lin_v"]["b"]
  qn = _dot(xtp, lp["query"]["w"]) + lp["query"]["b"]
  n = xn.shape[0]
  x32 = xn[:, :32]
  zk = jnp.zeros((n, s["Kp"] - s["K"]), F32)
  zv = jnp.zeros((n, s["Vp"] - s["T"]), F32)
  srow = jnp.concatenate([x32, kn, zk, vn, zv], axis=1)
  drow = jnp.concatenate([x32, qn, zk], axis=1)
  return srow, drow


def _p_node(t2, nf, mnf, vnf, freqs, params):
  n = nf.shape[0]
  nmlp = params["node_mlp"]
  lp0 = params["layers"][0]
  s0 = _layer_sizes(0)

  def body(t2_ref, nf_ref, mnf_ref, vnf_ref, fr_ref, w1, b1, w2, b2, w3, b3,
           wi, bi, wq, bq, wk, bk, wv, bv,
           nemb_ref, x0_ref, s_ref, d_ref):
    ang = t2_ref[...] * fr_ref[...]
    emb = jnp.concatenate([jnp.sin(ang), jnp.cos(ang)], axis=1)
    x = nf_ref[...]
    xln = (x - mnf_ref[...]) / jnp.sqrt(vnf_ref[...] + 1e-5)
    h = jnp.concatenate([emb, xln], axis=1)
    h = jnp.maximum(_dot(h, w1[...]) + b1[...], 0.0)
    h = jnp.maximum(_dot(h, w2[...]) + b2[...], 0.0)
    x0 = _dot(h, w3[...]) + b3[...]
    nemb_ref[...] = emb
    x0_ref[...] = x0
    xtp = _dot(x0, wi[...]) + bi[...]
    kn = _dot(xtp, wk[...]) + bk[...]
    vn = _dot(xtp, wv[...]) + bv[...]
    qn = _dot(xtp, wq[...]) + bq[...]
    nb = x0.shape[0]
    zk = jnp.zeros((nb, s0["Kp"] - s0["K"]), F32)
    zv = jnp.zeros((nb, s0["Vp"] - s0["T"]), F32)
    s_ref[...] = jnp.concatenate([x0, kn, zk, vn, zv], axis=1)
    d_ref[...] = jnp.concatenate([x0, qn, zk], axis=1)

  out_shape = [
      jax.ShapeDtypeStruct((n, 32), F32),
      jax.ShapeDtypeStruct((n, 32), F32),
      jax.ShapeDtypeStruct((n, s0["Ws"]), F32),
      jax.ShapeDtypeStruct((n, s0["Wd"]), F32),
  ]
  args = [t2, nf, mnf, vnf, freqs]
  for p in nmlp:
    args += [p["w"], p["b"][None, :]]
  for nm in ("input_linear", "query", "lin_k", "lin_v"):
    args += [lp0[nm]["w"], lp0[nm]["b"][None, :]]
  block = 2000
  in_specs = [pl.BlockSpec((block, 1), lambda i: (i, 0)),
              pl.BlockSpec((block, 256), lambda i: (i, 0)),
              pl.BlockSpec((block, 1), lambda i: (i, 0)),
              pl.BlockSpec((block, 1), lambda i: (i, 0)),
              pl.BlockSpec((1, 16), lambda i: (0, 0))]
  in_specs += [pl.BlockSpec(a.shape, lambda i: (0, 0)) for a in args[5:]]
  out_specs = [pl.BlockSpec((block, 32), lambda i: (i, 0)),
               pl.BlockSpec((block, 32), lambda i: (i, 0)),
               pl.BlockSpec((block, s0["Ws"]), lambda i: (i, 0)),
               pl.BlockSpec((block, s0["Wd"]), lambda i: (i, 0))]
  return pl.pallas_call(
      body, grid=(n // block,), in_specs=in_specs, out_specs=out_specs,
      out_shape=out_shape,
      compiler_params=pltpu.CompilerParams(
          dimension_semantics=("arbitrary",)))(*args)


def _p_edge(evec, temb, pe, ef, mef, vef, de, offs, coef, params, block=2000):
  e = ef.shape[0]
  emlp = params["edge_mlp"]

  def body(ev_ref, te_ref, pe_ref, ef_ref, mef_ref, vef_ref, de_ref,
           offs_ref, coef_ref, w1, b1, w2, b2, w3, b3, ed_ref, sh_ref):
    v3 = ev_ref[...]
    d = de_ref[...]
    dist = jnp.exp(coef_ref[...] * (d - offs_ref[...]) ** 2)
    x = ef_ref[...]
    xln = (x - mef_ref[...]) / jnp.sqrt(vef_ref[...] + 1e-5)
    h = jnp.concatenate([te_ref[...], dist, pe_ref[...], xln], axis=1)
    h = jnp.maximum(_dot(h, w1[...]) + b1[...], 0.0)
    h = jnp.maximum(_dot(h, w2[...]) + b2[...], 0.0)
    ed_ref[...] = _dot(h, w3[...]) + b3[...]
    nrm = v3 / (d + 1e-8)
    x1, y1, z1 = nrm[:, 0:1], nrm[:, 1:2], nrm[:, 2:3]
    s3 = np.float32(np.sqrt(3.0))
    s15 = np.float32(np.sqrt(15.0))
    s5h = np.float32(np.sqrt(5.0) / 2.0)
    sh = jnp.concatenate([
        jnp.ones_like(x1), s3 * x1, s3 * y1, s3 * z1,
        s15 * x1 * y1, s15 * y1 * z1, s5h * (3.0 * z1 * z1 - 1.0),
        s15 * x1 * z1, (s15 / 2.0) * (x1 * x1 - y1 * y1),
        jnp.zeros((v3.shape[0], 7), F32)], axis=1)
    sh_ref[...] = sh

  grid = (e // block,)
  full = lambda arr: pl.BlockSpec(arr.shape, lambda i: (0, 0))
  args = [evec, temb, pe, ef, mef, vef, de, offs, coef]
  wspecs = []
  for p in emlp:
    args += [p["w"], p["b"][None, :]]
  for a in args[9:]:
    wspecs.append(full(a))
  in_specs = [
      pl.BlockSpec((block, 3), lambda i: (i, 0)),
      pl.BlockSpec((block, 32), lambda i: (i, 0)),
      pl.BlockSpec((block, 16), lambda i: (i, 0)),
      pl.BlockSpec((block, 256), lambda i: (i, 0)),
      pl.BlockSpec((block, 1), lambda i: (i, 0)),
      pl.BlockSpec((block, 1), lambda i: (i, 0)),
      pl.BlockSpec((block, 1), lambda i: (i, 0)),
      pl.BlockSpec((1, 50), lambda i: (0, 0)),
      pl.BlockSpec((1, 1), lambda i: (0, 0)),
  ] + wspecs
  out_shape = [jax.ShapeDtypeStruct((e, 32), F32),
               jax.ShapeDtypeStruct((e, 16), F32)]
  out_specs = [pl.BlockSpec((block, 32), lambda i: (i, 0)),
               pl.BlockSpec((block, 16), lambda i: (i, 0))]
  return pl.pallas_call(
      body, grid=grid, in_specs=in_specs, out_specs=out_specs,
      out_shape=out_shape,
      compiler_params=pltpu.CompilerParams(
          dimension_semantics=("arbitrary",)))(*args)


def _k1_edge(ed, gs, gd, sh, lp, sizes, block=2000):
  e = ed.shape[0]
  s = sizes
  K, T, Kp, Ws, Wd = s["K"], s["T"], s["Kp"], s["Ws"], s["Wd"]

  def body(ed_ref, gs_ref, gd_ref, sh_ref,
           k1w, k1b, k2w, k2b, k3w, k3b, skw, skb,
           v1w, v1b, v2w, v2b, v3w, v3b, svw, svb,
           a_ref, v_ref):
    attr = jnp.concatenate(
        [ed_ref[...], gs_ref[:, :32], gd_ref[:, :32]], axis=1)
    sh9 = sh_ref[:, :9]
    h = jnp.maximum(_dot(attr, k1w[...]) + k1b[...], 0.0)
    h = jnp.maximum(_dot(h, k2w[...]) + k2b[...], 0.0)
    wk = _dot(h, k3w[...]) + k3b[...]
    shk = _dot(sh9, skw[...]) + skb[...]
    kk = gs_ref[:, 32:32 + K] * shk * wk
    a_ref[...] = jnp.sum(gd_ref[:, 32:32 + K] * kk, axis=1, keepdims=True)
    hv = jnp.maximum(_dot(attr, v1w[...]) + v1b[...], 0.0)
    hv = jnp.maximum(_dot(hv, v2w[...]) + v2b[...], 0.0)
    wv = _dot(hv, v3w[...]) + v3b[...]
    shv = _dot(sh9, svw[...]) + svb[...]
    vv = gs_ref[:, 32 + Kp:32 + Kp + T] * shv * wv
    nb = vv.shape[0]
    v_ref[...] = jnp.concatenate(
        [vv, jnp.zeros((nb, _VW - 1 - T), F32), jnp.ones((nb, 1), F32)],
        axis=1)

  args = [ed, gs, gd, sh]
  for p in lp["fc_key"]:
    args += [p["w"], p["b"][None, :]]
  args += [lp["lin_shk"]["w"], lp["lin_shk"]["b"][None, :]]
  for p in lp["fc_value"]:
    args += [p["w"], p["b"][None, :]]
  args += [lp["lin_shv"]["w"], lp["lin_shv"]["b"][None, :]]
  in_specs = [
      pl.BlockSpec((block, 32), lambda i: (i, 0)),
      pl.BlockSpec((block, Ws), lambda i: (i, 0)),
      pl.BlockSpec((block, Wd), lambda i: (i, 0)),
      pl.BlockSpec((block, 16), lambda i: (i, 0)),
  ] + [pl.BlockSpec(a.shape, lambda i: (0, 0)) for a in args[4:]]
  out_shape = [jax.ShapeDtypeStruct((e, 1), F32),
               jax.ShapeDtypeStruct((e, _VW), F32)]
  out_specs = [pl.BlockSpec((block, 1), lambda i: (i, 0)),
               pl.BlockSpec((block, _VW), lambda i: (i, 0))]
  return pl.pallas_call(
      body, grid=(e // block,), in_specs=in_specs, out_specs=out_specs,
      out_shape=out_shape,
      compiler_params=pltpu.CompilerParams(
          dimension_semantics=("arbitrary",)))(*args)


def _k4_scale(v, e2, block=2000):
  e = v.shape[0]

  def body(v_ref, e_ref, o_ref):
    o_ref[...] = v_ref[...] * e_ref[...]

  return pl.pallas_call(
      body, grid=(e // block,),
      in_specs=[pl.BlockSpec((block, _VW), lambda i: (i, 0)),
                pl.BlockSpec((block, 1), lambda i: (i, 0))],
      out_specs=pl.BlockSpec((block, _VW), lambda i: (i, 0)),
      out_shape=jax.ShapeDtypeStruct((e, _VW), F32),
      compiler_params=pltpu.CompilerParams(
          dimension_semantics=("arbitrary",)))(v, e2)


def _k6_node(acc2, x, lp, sizes, next_lp, next_sizes, params, block=2000):
  n = x.shape[0]
  s = sizes
  T, dout, din = s["T"], s["dout"], s["din"]
  final = next_lp is None
  nb_grid = (n // block,)
  sem = pltpu.CompilerParams(dimension_semantics=("arbitrary",))

  def body_o(acc_ref, x_ref, wo, bo, o_ref):
    acc = acc_ref[0] + acc_ref[1]
    z = acc[:, _VW - 1:_VW]
    msg = acc[:, :T] / (z + 1e-12)
    o = _dot(msg, wo[...]) + bo[...]
    xin = x_ref[...]
    if dout > din:
      xin = jnp.concatenate(
          [xin, jnp.zeros((xin.shape[0], dout - din), F32)], axis=1)
    o_ref[...] = o + xin

  o = pl.pallas_call(
      body_o, grid=nb_grid,
      in_specs=[pl.BlockSpec((2, block, _VW), lambda i: (0, i, 0)),
                pl.BlockSpec((block, din), lambda i: (i, 0)),
                pl.BlockSpec((T, dout), lambda i: (0, 0)),
                pl.BlockSpec((1, dout), lambda i: (0, 0))],
      out_specs=pl.BlockSpec((block, dout), lambda i: (i, 0)),
      out_shape=jax.ShapeDtypeStruct((n, dout), F32),
      compiler_params=sem)(
          acc2, x, lp["output_linear"]["w"],
          lp["output_linear"]["b"][None, :])

  m = o.mean(axis=0, keepdims=True)
  v = o.var(axis=0, keepdims=True)

  def body_t(*refs):
    if final:
      (o_ref, m_ref, v_ref, wa, ba, wb, bb, wc, bc, out_ref) = refs
    else:
      (o_ref, m_ref, v_ref, wi, bi, wq, bq, wk, bk, wv, bv,
       x1_ref, s_ref, d_ref) = refs
    xn = (o_ref[...] - m_ref[...]) / jnp.sqrt(v_ref[...] + 1e-5)
    if final:
      ha = _dot(xn, wa[...]) + ba[...]
      hb = _dot(xn, wb[...]) + bb[...]
      out_ref[...] = _dot(ha * hb, wc[...]) + bc[...]
    else:
      ns = next_sizes
      xtp = _dot(xn, wi[...]) + bi[...]
      kn = _dot(xtp, wk[...]) + bk[...]
      vn = _dot(xtp, wv[...]) + bv[...]
      qn = _dot(xtp, wq[...]) + bq[...]
      nb = xn.shape[0]
      zk = jnp.zeros((nb, ns["Kp"] - ns["K"]), F32)
      zv = jnp.zeros((nb, ns["Vp"] - ns["T"]), F32)
      x1_ref[...] = xn
      s_ref[...] = jnp.concatenate([xn[:, :32], kn, zk, vn, zv], axis=1)
      d_ref[...] = jnp.concatenate([xn[:, :32], qn, zk], axis=1)

  args = [o, m, v]
  if final:
    for nm in ("final_a", "final_b", "final_c"):
      args += [params[nm]["w"], params[nm]["b"][None, :]]
    out_shape = jax.ShapeDtypeStruct((n, 6), F32)
    out_specs = pl.BlockSpec((block, 6), lambda i: (i, 0))
  else:
    ns = next_sizes
    for nm in ("input_linear", "query", "lin_k", "lin_v"):
      args += [next_lp[nm]["w"], next_lp[nm]["b"][None, :]]
    out_shape = [jax.ShapeDtypeStruct((n, dout), F32),
                 jax.ShapeDtypeStruct((n, ns["Ws"]), F32),
                 jax.ShapeDtypeStruct((n, ns["Wd"]), F32)]
    out_specs = [pl.BlockSpec((block, dout), lambda i: (i, 0)),
                 pl.BlockSpec((block, ns["Ws"]), lambda i: (i, 0)),
                 pl.BlockSpec((block, ns["Wd"]), lambda i: (i, 0))]
  in_specs = [pl.BlockSpec((block, dout), lambda i: (i, 0))]
  in_specs += [pl.BlockSpec(a.shape, lambda i: (0, 0)) for a in args[1:]]
  return pl.pallas_call(
      body_t, grid=nb_grid, in_specs=in_specs, out_specs=out_specs,
      out_shape=out_shape, compiler_params=sem)(*args)


def _sc_gather(e_edges, tables):
  epw = e_edges // _NW
  nch = epw // _CH
  ntab = len(tables)
  mesh = plsc.VectorSubcoreMesh(core_axis_name="c", subcore_axis_name="s")
  out_type = tuple(
      jax.ShapeDtypeStruct((e_edges, w), F32) for (_, _, w) in tables)
  scratch = [pltpu.VMEM((nch, _CH), I32), pltpu.VMEM((nch, _CH), I32)]
  scratch += [pltpu.VMEM((_CH, w), F32) for (_, _, w) in tables]
  scratch += [pltpu.SemaphoreType.DMA]

  @functools.partial(pl.kernel, out_type=out_type, mesh=mesh,
                     scratch_types=scratch, compiler_params=_SC_PARAMS)
  def g(src2, dst2, *rest):
    tabs = rest[:ntab]
    outs = rest[ntab:2 * ntab]
    si, di = rest[2 * ntab], rest[2 * ntab + 1]
    bufs = rest[2 * ntab + 2:2 * ntab + 2 + ntab]
    sem = rest[-1]
    c = lax.axis_index("c")
    sid = lax.axis_index("s")
    wid = sid * _NC + c
    pltpu.sync_copy(src2.at[pl.ds(wid * nch, nch)], si)
    pltpu.sync_copy(dst2.at[pl.ds(wid * nch, nch)], di)

    def step(j, carry):
      for t in range(ntab):
        idxrow = (di if tables[t][0] else si).at[j]
        pltpu.async_copy(tabs[t].at[idxrow], bufs[t], sem).wait()
        pltpu.sync_copy(bufs[t],
                        outs[t].at[pl.ds(wid * epw + j * _CH, _CH)])
      return carry

    lax.fori_loop(0, nch, step, 0)

  return g


def _sc_segmax_exp(e_edges, n_nodes):
  npad = ((n_nodes + _NS * _LANES - 1) // (_NS * _LANES)) * (_NS * _LANES)
  slc = npad // _NS
  ept = e_edges // _NS
  epw = e_edges // _NW
  mesh = plsc.VectorSubcoreMesh(core_axis_name="c", subcore_axis_name="s")
  scratch = [
      pltpu.VMEM((npad,), F32),
      pltpu.VMEM((ept,), F32),
      pltpu.VMEM((ept,), I32),
      pltpu.VMEM((epw,), F32),
      pltpu.VMEM((_NS, slc), F32),
      pltpu.VMEM((slc,), F32),
      pltpu.VMEM_SHARED((_NS, npad), F32),
      pltpu.VMEM_SHARED((npad,), F32),
  ]

  @functools.partial(
      pl.kernel, out_type=jax.ShapeDtypeStruct((e_edges,), F32), mesh=mesh,
      scratch_types=scratch, compiler_params=_SC_PARAMS)
  def k(a_hbm, d_hbm, e_hbm, maxarr, abuf, ibuf, ebuf, bufr, bufred,
        spm16, spmf):
    c = lax.axis_index("c")
    sid = lax.axis_index("s")
    wid = sid * _NC + c
    iota = lax.iota(I32, _LANES)
    neg = jnp.full((_LANES,), _NEG, F32)

    def ini(j, carry):
      maxarr[pl.ds(j * _LANES, _LANES)] = neg
      return carry

    lax.fori_loop(0, npad // _LANES, ini, 0)

    pltpu.sync_copy(a_hbm.at[pl.ds(sid * ept, ept)], abuf)
    pltpu.sync_copy(d_hbm.at[pl.ds(sid * ept, ept)], ibuf)

    def mx(j, carry):
      off = j * _LANES
      sk, sv = plsc.sort_key_val(ibuf[pl.ds(off, _LANES)],
                                 abuf[pl.ds(off, _LANES)])
      for sft in (1, 2, 4, 8):
        im = jnp.maximum(iota - sft, 0)
        pk = jnp.take_along_axis(sk, im, axis=0, mode="promise_in_bounds")
        pv = jnp.take_along_axis(sv, im, axis=0, mode="promise_in_bounds")
        ok = (iota >= sft) & (pk == sk)
        sv = jnp.where(ok, jnp.maximum(sv, pv), sv)
      nk = jnp.take_along_axis(sk, jnp.minimum(iota + 1, _LANES - 1),
                               axis=0, mode="promise_in_bounds")
      last = (nk != sk) | (iota == _LANES - 1)
      cur = plsc.load_gather(maxarr, [sk], mask=last)
      plsc.store_scatter(maxarr, [sk], jnp.maximum(sv, cur), mask=last)
      return carry

    lax.fori_loop(0, ept // _LANES, mx, 0)

    pltpu.sync_copy(maxarr, spm16.at[sid])
    plsc.subcore_barrier()
    base = sid * slc
    for t in range(_NS):
      pltpu.sync_copy(spm16.at[t, pl.ds(base, slc)], bufr.at[t])

    def rd(g, carry):
      off = g * _LANES
      m = bufr[0, pl.ds(off, _LANES)]
      for t in range(1, _NS):
        m = jnp.maximum(m, bufr[t, pl.ds(off, _LANES)])
      m = jnp.where(m < -1e38, 0.0, m)
      bufred[pl.ds(off, _LANES)] = m
      return carry

    lax.fori_loop(0, slc // _LANES, rd, 0)
    pltpu.sync_copy(bufred, spmf.at[pl.ds(base, slc)])
    plsc.subcore_barrier()
    pltpu.sync_copy(spmf, maxarr)

    pltpu.sync_copy(a_hbm.at[pl.ds(wid * epw, epw)], abuf.at[pl.ds(0, epw)])
    pltpu.sync_copy(d_hbm.at[pl.ds(wid * epw, epw)], ibuf.at[pl.ds(0, epw)])
    nvec = (epw + _LANES - 1) // _LANES

    def ex(j, carry):
      off = jnp.minimum(j * _LANES, epw - _LANES)
      k16 = ibuf[pl.ds(off, _LANES)]
      a16 = abuf[pl.ds(off, _LANES)]
      mx16 = plsc.load_gather(maxarr, [k16])
      ebuf[pl.ds(off, _LANES)] = jnp.exp(a16 - mx16)
      return carry

    lax.fori_loop(0, nvec, ex, 0)
    pltpu.sync_copy(ebuf, e_hbm.at[pl.ds(wid * epw, epw)])

  return k


def _sc_scatter_add(e_edges, n_nodes):
  epw = e_edges // _NW
  nch = epw // _CH
  rpt = n_nodes // _NS
  mesh = plsc.VectorSubcoreMesh(core_axis_name="c", subcore_axis_name="s")
  scratch = [
      pltpu.VMEM((rpt, _VW), F32),
      pltpu.VMEM((_CH, _VW), F32),
      pltpu.VMEM((nch, _CH), I32),
      pltpu.VMEM_SHARED((n_nodes, _VW), F32),
  ]

  @functools.partial(
      pl.kernel, mesh=mesh,
      out_type=jax.ShapeDtypeStruct((_NC, n_nodes, _VW), F32),
      scratch_types=scratch, compiler_params=_SC_PARAMS)
  def k(aug_hbm, d2_hbm, out_hbm, zbuf, rbuf, idx2, spacc):
    c = lax.axis_index("c")
    sid = lax.axis_index("s")
    wid = sid * _NC + c
    zv = jnp.zeros((_LANES,), F32)

    def z1(j, carry):
      for cc in range(_VW // _LANES):
        zbuf[j, pl.ds(cc * _LANES, _LANES)] = zv
      return carry

    lax.fori_loop(0, rpt, z1, 0)
    pltpu.sync_copy(zbuf, spacc.at[pl.ds(sid * rpt, rpt)])
    pltpu.sync_copy(d2_hbm.at[pl.ds(wid * nch, nch)], idx2)
    plsc.subcore_barrier()

    def st(j, carry):
      pltpu.sync_copy(aug_hbm.at[pl.ds(wid * epw + j * _CH, _CH)], rbuf)
      pltpu.sync_copy(rbuf, spacc.at[idx2.at[j]], add=True)
      return carry

    lax.fori_loop(0, nch, st, 0)
    plsc.subcore_barrier()
    pltpu.sync_copy(spacc.at[pl.ds(sid * rpt, rpt)], zbuf)
    pltpu.sync_copy(zbuf, out_hbm.at[c, pl.ds(sid * rpt, rpt)])

  return k


def kernel(node_features, edge_index, edge_features, edge_vectors, t,
           pos_emb, params):
  n = node_features.shape[0]
  e = edge_index.shape[1]
  src = edge_index[0].astype(I32)
  dst = edge_index[1].astype(I32)
  src2 = src.reshape(e // _CH, _CH)
  dst2 = dst.reshape(e // _CH, _CH)
  t2 = t[:, None].astype(F32)

  sizes = [_layer_sizes(i) for i in range(4)]

  freqs = jnp.exp(-np.log(10000.0) * jnp.arange(16, dtype=F32) / 15)[None, :]
  mnf = node_features.mean(-1, keepdims=True)
  vnf = node_features.var(-1, keepdims=True)
  mef = edge_features.mean(-1, keepdims=True)
  vef = edge_features.var(-1, keepdims=True)
  de = jnp.linalg.norm(edge_vectors, axis=-1)[:, None]
  offs = jnp.linspace(0.0, 50 ** 0.5, 50)
  coef = (-0.5 / (offs[1] - offs[0]) ** 2).astype(F32).reshape(1, 1)
  offs = offs.astype(F32)[None, :]

  nemb, x0, s_tab, d_tab = _p_node(t2, node_features, mnf, vnf, freqs, params)

  g0 = _sc_gather(e, [(False, n, 32),
                      (False, n, sizes[0]["Ws"]),
                      (True, n, sizes[0]["Wd"])])
  temb, gs, gd = g0(src2, dst2, nemb, s_tab, d_tab)

  ed, sh = _p_edge(edge_vectors.astype(F32), temb, pos_emb.astype(F32),
                   edge_features.astype(F32), mef, vef, de, offs, coef,
                   params)

  x = x0
  out = None
  for i in range(4):
    lp = params["layers"][i]
    s = sizes[i]
    a1, v = _k1_edge(ed, gs, gd, sh, lp, s)
    e_edge = _sc_segmax_exp(e, n)(a1.reshape(e), dst)
    aug = _k4_scale(v, e_edge.reshape(e, 1))
    acc2 = _sc_scatter_add(e, n)(aug, dst2)
    if i < 3:
      nxt = params["layers"][i + 1]
      x, s_tab, d_tab = _k6_node(acc2, x, lp, s, nxt, sizes[i + 1], params)
      gi = _sc_gather(e, [(False, n, sizes[i + 1]["Ws"]),
                          (True, n, sizes[i + 1]["Wd"])])
      gs, gd = gi(src2, dst2, s_tab, d_tab)
    else:
      out = _k6_node(acc2, x, lp, s, None, None, params)
  return out
